# Initial kernel scaffold; baseline (speedup 1.0000x reference)
import jax
import jax.numpy as jnp
from jax.experimental import pallas as pl


def kernel(node_position, node_in, node_out, relation, node2graph, bead2residue, residue_type):
    raise NotImplementedError("write your pallas kernel here")


# speedup vs baseline: 1.0222x; 1.0222x over previous
import functools

import jax
import jax.numpy as jnp
from jax.experimental import pallas as pl
from jax.experimental.pallas import tpu as pltpu

_NUM_RES = 21
_NUM_REL = 7
_MAXD = 10
_B = 64
_F = _NUM_RES * 2 + _NUM_REL + _MAXD + 1 + 1
_RB = 6400


def _feat_body(ti_ref, to_ref, r_ref, sd_ref, dist_ref, delta_ref, cume_ref,
               feat_ref, off_ref):
    col = jax.lax.broadcasted_iota(jnp.int32, (_RB, _F), 1)
    ti = ti_ref[0, 0, :][:, None]
    to = to_ref[0, 0, :][:, None]
    rr = r_ref[0, 0, :][:, None]
    sd = sd_ref[0, 0, :][:, None]
    hit = (col == ti) | (col == to + _NUM_RES) | (col == rr + 2 * _NUM_RES) | (
        col == sd + 2 * _NUM_RES + _NUM_REL)
    f = jnp.where(hit, jnp.float32(1.0), jnp.float32(0.0))
    f = jnp.where(col == _F - 1, dist_ref[0, 0, :][:, None], f)
    feat_ref[...] = f
    d0 = pl.program_id(0) * _RB
    dmat = d0 + jax.lax.broadcasted_iota(jnp.int32, (_RB, _B), 0)
    ge = dmat >= cume_ref[:][None, :]
    off_ref[0, 0, :] = jnp.sum(jnp.where(ge, delta_ref[:][None, :], 0), axis=1)


@jax.jit
def _feature_call(ti, to, r, sd, dist, delta, cume):
    e = ti.shape[0]
    grid = e // _RB
    row_spec = pl.BlockSpec((1, 1, _RB), lambda i: (i, 0, 0))
    small_spec = pl.BlockSpec((_B,), lambda i: (0,))
    r3 = lambda x: x.reshape(grid, 1, _RB)
    feat, off3 = pl.pallas_call(
        _feat_body,
        grid=(grid,),
        in_specs=[row_spec, row_spec, row_spec, row_spec, row_spec,
                  small_spec, small_spec],
        out_specs=[pl.BlockSpec((_RB, _F), lambda i: (i, 0)), row_spec],
        out_shape=[
            jax.ShapeDtypeStruct((e, _F), jnp.float32),
            jax.ShapeDtypeStruct((grid, 1, _RB), jnp.int32),
        ],
    )(r3(ti), r3(to), r3(r), r3(sd), r3(dist), delta, cume)
    return feat, off3.reshape(e)


def kernel(node_position, node_in, node_out, relation, node2graph,
           bead2residue, residue_type):
    e2g = node2graph[node_in]
    order = jnp.argsort(e2g)
    ni = node_in[order]
    no = node_out[order]
    r = relation[order]
    num_edges = jnp.bincount(e2g, length=_B).astype(jnp.int32)
    num_nodes = jnp.bincount(node2graph, length=_B)
    cumn = jnp.cumsum(num_nodes)
    offtable = (cumn - num_nodes).astype(jnp.int32)
    delta = offtable - jnp.concatenate([jnp.zeros((1,), jnp.int32), offtable[:-1]])
    cume = (jnp.cumsum(num_edges) - num_edges).astype(jnp.int32)
    ri = bead2residue[ni]
    ro = bead2residue[no]
    ti = residue_type[ri]
    to = residue_type[ro]
    sd = jnp.clip(jnp.abs(ri - ro), 0, _MAXD).astype(jnp.int32)
    diff = node_position[ni] - node_position[no]
    dist = jnp.sqrt(jnp.sum(diff * diff, axis=-1) + 1e-12)
    feat, offsets = _feature_call(ti, to, r, sd, dist, delta, cume)
    edge_list = jnp.stack([ni, no, r], axis=-1)
    return edge_list, num_edges, offsets, feat


# speedup vs baseline: 1.1143x; 1.0901x over previous
import functools

import jax
import jax.numpy as jnp
from jax import lax
from jax.experimental import pallas as pl
from jax.experimental.pallas import tpu as pltpu
from jax.experimental.pallas import tpu_sc as plsc

_NUM_RES = 21
_NUM_REL = 7
_MAXD = 10
_B = 64
_F = _NUM_RES * 2 + _NUM_REL + _MAXD + 1 + 1
_RB = 6400

_NW = 32
_NL = 16
_E = 800000
_S = 1568
_CH = _S * _NL
_EPAD = _S * _NL * _NW
_N = 50000
_NPAD = 50176
_NCH = _NPAD // _NW
_KB = 80


_MESH = plsc.VectorSubcoreMesh(core_axis_name="c", subcore_axis_name="s")
_IOTA = functools.partial(lax.broadcasted_iota, jnp.int32, (_NL,), 0)


def _wid():
    return lax.axis_index("c") * _NL + lax.axis_index("s")


@functools.partial(
    pl.kernel,
    out_type=[
        jax.ShapeDtypeStruct((_EPAD,), jnp.int32),
        jax.ShapeDtypeStruct((_NW, _KB, _NL), jnp.int32),
        jax.ShapeDtypeStruct((_NW, _KB), jnp.int32),
        jax.ShapeDtypeStruct((_NW, _KB), jnp.int32),
    ],
    mesh=_MESH,
    compiler_params=pltpu.CompilerParams(needs_layout_passes=False),
    scratch_types=[
        pltpu.VMEM((_NPAD,), jnp.int32),
        pltpu.VMEM((_CH,), jnp.int32),
        pltpu.VMEM((_CH,), jnp.int32),
        pltpu.VMEM((_KB, _NL), jnp.int32),
        pltpu.VMEM((_KB, _NL), jnp.int32),
        pltpu.VMEM((_NL, _KB), jnp.int32),
        pltpu.VMEM((_NL, _KB), jnp.int32),
        pltpu.VMEM((_KB,), jnp.int32),
        pltpu.VMEM((_KB,), jnp.int32),
    ],
)
def _pass_a(nin_hbm, n2g_hbm, e2g_hbm, histe_hbm, re_hbm, rn_hbm,
            n2g_v, nin_v, e2g_v, he_v, hn_v, heT_v, hnT_v, re_v, rn_v):
    w = _wid()
    lane = _IOTA()
    ones = jnp.ones((_NL,), jnp.int32)
    zeros = jnp.zeros((_NL,), jnp.int32)
    pltpu.sync_copy(n2g_hbm, n2g_v)
    pltpu.sync_copy(nin_hbm.at[pl.ds(w * _CH, _CH)], nin_v)

    def zrow(k, _):
        he_v[k, :] = zeros
        hn_v[k, :] = zeros
        return 0
    lax.fori_loop(0, _KB, zrow, 0)

    def estep(s, _):
        idx = lane * _S + s
        ni = plsc.load_gather(nin_v, [idx])
        k = plsc.load_gather(n2g_v, [ni])
        plsc.store_scatter(e2g_v, [idx], k)
        plsc.addupdate_scatter(he_v, [k, lane], ones)
        return 0
    lax.fori_loop(0, _S, estep, 0)
    pltpu.sync_copy(e2g_v, e2g_hbm.at[pl.ds(w * _CH, _CH)])

    def nstep(s, _):
        k = n2g_v[pl.ds(w * _NCH + s * _NL, _NL)]
        plsc.addupdate_scatter(hn_v, [k, lane], ones)
        return 0
    lax.fori_loop(0, _NCH // _NL, nstep, 0)

    def tpose(k, _):
        kv = jnp.full((_NL,), k, jnp.int32)
        plsc.store_scatter(heT_v, [lane, kv], he_v[k, :])
        plsc.store_scatter(hnT_v, [lane, kv], hn_v[k, :])
        return 0
    lax.fori_loop(0, _KB, tpose, 0)
    for c in range(_KB // _NL):
        acc_e = jnp.zeros((_NL,), jnp.int32)
        acc_n = jnp.zeros((_NL,), jnp.int32)
        for l in range(_NL):
            acc_e = acc_e + heT_v[l, pl.ds(c * _NL, _NL)]
            acc_n = acc_n + hnT_v[l, pl.ds(c * _NL, _NL)]
        re_v[pl.ds(c * _NL, _NL)] = acc_e
        rn_v[pl.ds(c * _NL, _NL)] = acc_n
    pltpu.sync_copy(he_v, histe_hbm.at[w])
    pltpu.sync_copy(re_v, re_hbm.at[w])
    pltpu.sync_copy(rn_v, rn_hbm.at[w])


_NRING = 8
_RW = 128
_NSTEP = _RW // _NL


@functools.partial(
    pl.kernel,
    out_type=[
        jax.ShapeDtypeStruct((_EPAD,), jnp.int32),
        jax.ShapeDtypeStruct((_EPAD,), jnp.int32),
        jax.ShapeDtypeStruct((_EPAD,), jnp.int32),
        jax.ShapeDtypeStruct((_B,), jnp.int32),
        jax.ShapeDtypeStruct((_B,), jnp.int32),
        jax.ShapeDtypeStruct((_B,), jnp.int32),
    ],
    mesh=_MESH,
    compiler_params=pltpu.CompilerParams(needs_layout_passes=False),
    scratch_types=[
        pltpu.VMEM((_NW, _KB), jnp.int32),
        pltpu.VMEM((_KB, _NL), jnp.int32),
        pltpu.VMEM((_KB, _NL), jnp.int32),
        pltpu.VMEM((_KB,), jnp.int32),
        pltpu.VMEM((_KB,), jnp.int32),
        pltpu.VMEM((_KB,), jnp.int32),
        pltpu.VMEM((_CH,), jnp.int32),
        pltpu.VMEM((_CH,), jnp.int32),
        pltpu.VMEM((_CH,), jnp.int32),
        pltpu.VMEM((_CH,), jnp.int32),
        pltpu.VMEM((_NRING, _RW), jnp.int32),
        pltpu.VMEM((_NRING, _RW), jnp.int32),
        pltpu.VMEM((_NRING, _RW), jnp.int32),
        pltpu.VMEM((_NRING, _RW), jnp.int32),
        pltpu.VMEM((3 * _RW,), jnp.int32),
        pltpu.SemaphoreType.DMA,
    ],
)
def _pass_b(e2g_hbm, histe_hbm, re_hbm, rn_hbm, nin_hbm, nout_hbm, rel_hbm,
            sni_hbm, sno_hbm, sr_hbm, ne_hbm, cume_hbm, delta_hbm,
            re_all, he_v, cur_v, pb_v, st1_v, st2_v,
            e2g_c, nin_c, nout_c, rel_c,
            dring, niring, noring, rring, drain_v, sem):
    w = _wid()
    lane = _IOTA()
    pltpu.sync_copy(re_hbm, re_all)
    pltpu.sync_copy(histe_hbm.at[w], he_v)
    pltpu.sync_copy(e2g_hbm.at[pl.ds(w * _CH, _CH)], e2g_c)
    pltpu.sync_copy(nin_hbm.at[pl.ds(w * _CH, _CH)], nin_c)
    pltpu.sync_copy(nout_hbm.at[pl.ds(w * _CH, _CH)], nout_c)
    pltpu.sync_copy(rel_hbm.at[pl.ds(w * _CH, _CH)], rel_c)

    nch = _KB // _NL
    zeros = jnp.zeros((_NL,), jnp.int32)

    def wacc(wp, carry):
        tot = list(carry[:nch])
        spv = list(carry[nch:])
        for c in range(nch):
            row = re_all[wp, pl.ds(c * _NL, _NL)]
            tot[c] = tot[c] + row
            spv[c] = spv[c] + jnp.where(wp < w, row, 0)
        return tuple(tot) + tuple(spv)
    init = tuple([zeros] * (2 * nch))
    acc = lax.fori_loop(0, _NW, wacc, init)
    tot = acc[:nch]
    spv = acc[nch:]

    carry = jnp.int32(0)
    for c in range(nch):
        incl = jnp.cumsum(tot[c])
        pb_v[pl.ds(c * _NL, _NL)] = incl - tot[c] + carry + spv[c]
        carry = carry + jnp.sum(tot[c])

    def baserow(k, _):
        row = he_v[k, :]
        incl = jnp.cumsum(row)
        pb = plsc.load_gather(pb_v, [jnp.full((_NL,), k, jnp.int32)])
        cur_v[k, :] = incl - row + pb
        return 0
    lax.fori_loop(0, _KB, baserow, 0)

    @pl.when(w == 0)
    def _():
        carry_e = jnp.int32(0)
        for c in range(4):
            st1_v[pl.ds(c * _NL, _NL)] = tot[c]
            incl = jnp.cumsum(tot[c])
            st2_v[pl.ds(c * _NL, _NL)] = incl - tot[c] + carry_e
            carry_e = carry_e + jnp.sum(tot[c])
        pltpu.sync_copy(st1_v.at[pl.ds(0, _B)], ne_hbm)
        pltpu.sync_copy(st2_v.at[pl.ds(0, _B)], cume_hbm)
        pltpu.sync_copy(rn_hbm, re_all)

        def nacc(wp, carry):
            return tuple(carry[c] + re_all[wp, pl.ds(c * _NL, _NL)]
                         for c in range(4))
        ntot = lax.fori_loop(0, _NW, nacc, tuple([zeros] * 4))
        for c in range(4):
            st1_v[pl.ds(c * _NL, _NL)] = ntot[c]
        ii = _IOTA()
        for c in range(4):
            gidx = ii + (c * _NL - 1)
            shifted = plsc.load_gather(st1_v, [jnp.maximum(gidx, 0)])
            st2_v[pl.ds(c * _NL, _NL)] = jnp.where(gidx < 0, 0, shifted)
        pltpu.sync_copy(st2_v.at[pl.ds(0, _B)], delta_hbm)

    nrows = _S // _NSTEP

    def prow(r, _):
        slot = lax.rem(r, _NRING)

        @pl.when(r >= _NRING)
        def _():
            pltpu.make_async_copy(
                sni_hbm.at[pl.ds(0, 3 * _RW)], drain_v, sem).wait()

        for j in range(_NSTEP):
            idx = lane * _S + r * _NSTEP + j
            k = plsc.load_gather(e2g_c, [idx])
            d = plsc.load_gather(cur_v, [k, lane])
            plsc.store_scatter(cur_v, [k, lane], d + 1)
            dring[slot, pl.ds(j * _NL, _NL)] = d
            niring[slot, pl.ds(j * _NL, _NL)] = plsc.load_gather(nin_c, [idx])
            noring[slot, pl.ds(j * _NL, _NL)] = plsc.load_gather(nout_c, [idx])
            rring[slot, pl.ds(j * _NL, _NL)] = plsc.load_gather(rel_c, [idx])
        pltpu.async_copy(niring.at[slot], sni_hbm.at[dring.at[slot]], sem)
        pltpu.async_copy(noring.at[slot], sno_hbm.at[dring.at[slot]], sem)
        pltpu.async_copy(rring.at[slot], sr_hbm.at[dring.at[slot]], sem)
        return 0
    lax.fori_loop(0, nrows, prow, 0)

    def dr(i, _):
        pltpu.make_async_copy(
            sni_hbm.at[pl.ds(0, 3 * _RW)], drain_v, sem).wait()
        return 0
    lax.fori_loop(0, _NRING, dr, 0)


def _feat_body(ti_ref, to_ref, r_ref, sd_ref, dist_ref, delta_ref, cume_ref,
               feat_ref, off_ref):
    col = jax.lax.broadcasted_iota(jnp.int32, (_RB, _F), 1)
    ti = ti_ref[0, 0, :][:, None]
    to = to_ref[0, 0, :][:, None]
    rr = r_ref[0, 0, :][:, None]
    sd = sd_ref[0, 0, :][:, None]
    hit = (col == ti) | (col == to + _NUM_RES) | (col == rr + 2 * _NUM_RES) | (
        col == sd + 2 * _NUM_RES + _NUM_REL)
    f = jnp.where(hit, jnp.float32(1.0), jnp.float32(0.0))
    f = jnp.where(col == _F - 1, dist_ref[0, 0, :][:, None], f)
    feat_ref[...] = f
    d0 = pl.program_id(0) * _RB
    dmat = d0 + jax.lax.broadcasted_iota(jnp.int32, (_RB, _B), 0)
    ge = dmat >= cume_ref[:][None, :]
    off_ref[0, 0, :] = jnp.sum(jnp.where(ge, delta_ref[:][None, :], 0), axis=1)


@jax.jit
def _feature_call(ti, to, r, sd, dist, delta, cume):
    e = ti.shape[0]
    grid = e // _RB
    row_spec = pl.BlockSpec((1, 1, _RB), lambda i: (i, 0, 0))
    small_spec = pl.BlockSpec((_B,), lambda i: (0,))
    r3 = lambda x: x.reshape(grid, 1, _RB)
    feat, off3 = pl.pallas_call(
        _feat_body,
        grid=(grid,),
        in_specs=[row_spec, row_spec, row_spec, row_spec, row_spec,
                  small_spec, small_spec],
        out_specs=[pl.BlockSpec((_RB, _F), lambda i: (i, 0)), row_spec],
        out_shape=[
            jax.ShapeDtypeStruct((e, _F), jnp.float32),
            jax.ShapeDtypeStruct((grid, 1, _RB), jnp.int32),
        ],
    )(r3(ti), r3(to), r3(r), r3(sd), r3(dist), delta, cume)
    return feat, off3.reshape(e)


def kernel(node_position, node_in, node_out, relation, node2graph,
           bead2residue, residue_type):
    e = node_in.shape[0]
    epad = _EPAD - e
    sentinel = jnp.int32(_NPAD - 1)
    nin_p = jnp.concatenate([node_in, jnp.full((epad,), sentinel, jnp.int32)])
    nout_p = jnp.concatenate([node_out, jnp.full((epad,), sentinel, jnp.int32)])
    rel_p = jnp.concatenate([relation, jnp.zeros((epad,), jnp.int32)])
    n2g_p = jnp.concatenate(
        [node2graph, jnp.full((_NPAD - _N,), jnp.int32(_B), jnp.int32)])
    e2g_p, histe, re_sums, rn_sums = _pass_a(nin_p, n2g_p)
    sni, sno, sr, num_edges, cume, delta = _pass_b(
        e2g_p, histe, re_sums, rn_sums, nin_p, nout_p, rel_p)
    ni = sni[:e]
    no = sno[:e]
    r = sr[:e]
    ri = bead2residue[ni]
    ro = bead2residue[no]
    ti = residue_type[ri]
    to = residue_type[ro]
    sd = jnp.clip(jnp.abs(ri - ro), 0, _MAXD).astype(jnp.int32)
    diff = node_position[ni] - node_position[no]
    dist = jnp.sqrt(jnp.sum(diff * diff, axis=-1) + 1e-12)
    feat, offsets = _feature_call(ti, to, r, sd, dist, delta, cume)
    edge_list = jnp.stack([ni, no, r], axis=-1)
    return edge_list, num_edges, offsets, feat


# speedup vs baseline: 8.1149x; 7.2826x over previous
import functools

import jax
import jax.numpy as jnp
from jax import lax
from jax.experimental import pallas as pl
from jax.experimental.pallas import tpu as pltpu
from jax.experimental.pallas import tpu_sc as plsc

_NUM_RES = 21
_NUM_REL = 7
_MAXD = 10
_B = 64
_F = _NUM_RES * 2 + _NUM_REL + _MAXD + 1 + 1
_RB = 6400

_NW = 32
_NL = 16
_E = 800000
_S = 1568
_CH = _S * _NL
_EPAD = _S * _NL * _NW
_N = 50000
_NPAD = 50176
_NCH = _NPAD // _NW
_KB = 80


_MESH = plsc.VectorSubcoreMesh(core_axis_name="c", subcore_axis_name="s")
_IOTA = functools.partial(lax.broadcasted_iota, jnp.int32, (_NL,), 0)


def _wid():
    return lax.axis_index("c") * _NL + lax.axis_index("s")


@functools.partial(
    pl.kernel,
    out_type=[
        jax.ShapeDtypeStruct((_EPAD,), jnp.int32),
        jax.ShapeDtypeStruct((_NW, _KB, _NL), jnp.int32),
        jax.ShapeDtypeStruct((_NW, _KB), jnp.int32),
        jax.ShapeDtypeStruct((_NW, _KB), jnp.int32),
    ],
    mesh=_MESH,
    compiler_params=pltpu.CompilerParams(needs_layout_passes=False),
    scratch_types=[
        pltpu.VMEM((_NPAD,), jnp.int32),
        pltpu.VMEM((_CH,), jnp.int32),
        pltpu.VMEM((_CH,), jnp.int32),
        pltpu.VMEM((_KB, _NL), jnp.int32),
        pltpu.VMEM((_KB, _NL), jnp.int32),
        pltpu.VMEM((_NL, _KB), jnp.int32),
        pltpu.VMEM((_NL, _KB), jnp.int32),
        pltpu.VMEM((_KB,), jnp.int32),
        pltpu.VMEM((_KB,), jnp.int32),
    ],
)
def _pass_a(nin_hbm, n2g_hbm, e2g_hbm, histe_hbm, re_hbm, rn_hbm,
            n2g_v, nin_v, e2g_v, he_v, hn_v, heT_v, hnT_v, re_v, rn_v):
    w = _wid()
    lane = _IOTA()
    ones = jnp.ones((_NL,), jnp.int32)
    zeros = jnp.zeros((_NL,), jnp.int32)
    pltpu.sync_copy(n2g_hbm, n2g_v)
    pltpu.sync_copy(nin_hbm.at[pl.ds(w * _CH, _CH)], nin_v)

    def zrow(k, _):
        he_v[k, :] = zeros
        hn_v[k, :] = zeros
        return 0
    lax.fori_loop(0, _KB, zrow, 0)

    def estep(s, _):
        idx = lane * _S + s
        ni = plsc.load_gather(nin_v, [idx])
        k = plsc.load_gather(n2g_v, [ni])
        plsc.store_scatter(e2g_v, [idx], k)
        plsc.addupdate_scatter(he_v, [k, lane], ones)
        return 0
    lax.fori_loop(0, _S, estep, 0)
    pltpu.sync_copy(e2g_v, e2g_hbm.at[pl.ds(w * _CH, _CH)])

    def nstep(s, _):
        k = n2g_v[pl.ds(w * _NCH + s * _NL, _NL)]
        plsc.addupdate_scatter(hn_v, [k, lane], ones)
        return 0
    lax.fori_loop(0, _NCH // _NL, nstep, 0)

    def tpose(k, _):
        kv = jnp.full((_NL,), k, jnp.int32)
        plsc.store_scatter(heT_v, [lane, kv], he_v[k, :])
        plsc.store_scatter(hnT_v, [lane, kv], hn_v[k, :])
        return 0
    lax.fori_loop(0, _KB, tpose, 0)
    for c in range(_KB // _NL):
        acc_e = jnp.zeros((_NL,), jnp.int32)
        acc_n = jnp.zeros((_NL,), jnp.int32)
        for l in range(_NL):
            acc_e = acc_e + heT_v[l, pl.ds(c * _NL, _NL)]
            acc_n = acc_n + hnT_v[l, pl.ds(c * _NL, _NL)]
        re_v[pl.ds(c * _NL, _NL)] = acc_e
        rn_v[pl.ds(c * _NL, _NL)] = acc_n
    pltpu.sync_copy(he_v, histe_hbm.at[w])
    pltpu.sync_copy(re_v, re_hbm.at[w])
    pltpu.sync_copy(rn_v, rn_hbm.at[w])


_NRING = 8
_RW = 128
_NSTEP = _RW // _NL


@functools.partial(
    pl.kernel,
    out_type=[
        jax.ShapeDtypeStruct((_EPAD,), jnp.int32),
        jax.ShapeDtypeStruct((_EPAD,), jnp.int32),
        jax.ShapeDtypeStruct((_EPAD,), jnp.int32),
        jax.ShapeDtypeStruct((_B,), jnp.int32),
        jax.ShapeDtypeStruct((_B,), jnp.int32),
        jax.ShapeDtypeStruct((_B,), jnp.int32),
    ],
    mesh=_MESH,
    compiler_params=pltpu.CompilerParams(needs_layout_passes=False),
    scratch_types=[
        pltpu.VMEM((_NW, _KB), jnp.int32),
        pltpu.VMEM((_KB, _NL), jnp.int32),
        pltpu.VMEM((_KB, _NL), jnp.int32),
        pltpu.VMEM((_KB,), jnp.int32),
        pltpu.VMEM((_KB,), jnp.int32),
        pltpu.VMEM((_KB,), jnp.int32),
        pltpu.VMEM((_CH,), jnp.int32),
        pltpu.VMEM((_CH,), jnp.int32),
        pltpu.VMEM((_CH,), jnp.int32),
        pltpu.VMEM((_CH,), jnp.int32),
        pltpu.VMEM((_NRING, _RW), jnp.int32),
        pltpu.VMEM((_NRING, _RW), jnp.int32),
        pltpu.VMEM((_NRING, _RW), jnp.int32),
        pltpu.VMEM((_NRING, _RW), jnp.int32),
        pltpu.VMEM((3 * _RW,), jnp.int32),
        pltpu.SemaphoreType.DMA,
    ],
)
def _pass_b(e2g_hbm, histe_hbm, re_hbm, rn_hbm, nin_hbm, nout_hbm, rel_hbm,
            sni_hbm, sno_hbm, sr_hbm, ne_hbm, cume_hbm, delta_hbm,
            re_all, he_v, cur_v, pb_v, st1_v, st2_v,
            e2g_c, nin_c, nout_c, rel_c,
            dring, niring, noring, rring, drain_v, sem):
    w = _wid()
    lane = _IOTA()
    pltpu.sync_copy(re_hbm, re_all)
    pltpu.sync_copy(histe_hbm.at[w], he_v)
    pltpu.sync_copy(e2g_hbm.at[pl.ds(w * _CH, _CH)], e2g_c)
    pltpu.sync_copy(nin_hbm.at[pl.ds(w * _CH, _CH)], nin_c)
    pltpu.sync_copy(nout_hbm.at[pl.ds(w * _CH, _CH)], nout_c)
    pltpu.sync_copy(rel_hbm.at[pl.ds(w * _CH, _CH)], rel_c)

    nch = _KB // _NL
    zeros = jnp.zeros((_NL,), jnp.int32)

    def wacc(wp, carry):
        tot = list(carry[:nch])
        spv = list(carry[nch:])
        for c in range(nch):
            row = re_all[wp, pl.ds(c * _NL, _NL)]
            tot[c] = tot[c] + row
            spv[c] = spv[c] + jnp.where(wp < w, row, 0)
        return tuple(tot) + tuple(spv)
    init = tuple([zeros] * (2 * nch))
    acc = lax.fori_loop(0, _NW, wacc, init)
    tot = acc[:nch]
    spv = acc[nch:]

    carry = jnp.int32(0)
    for c in range(nch):
        incl = jnp.cumsum(tot[c])
        pb_v[pl.ds(c * _NL, _NL)] = incl - tot[c] + carry + spv[c]
        carry = carry + jnp.sum(tot[c])

    def baserow(k, _):
        row = he_v[k, :]
        incl = jnp.cumsum(row)
        pb = plsc.load_gather(pb_v, [jnp.full((_NL,), k, jnp.int32)])
        cur_v[k, :] = incl - row + pb
        return 0
    lax.fori_loop(0, _KB, baserow, 0)

    @pl.when(w == 0)
    def _():
        carry_e = jnp.int32(0)
        for c in range(4):
            st1_v[pl.ds(c * _NL, _NL)] = tot[c]
            incl = jnp.cumsum(tot[c])
            st2_v[pl.ds(c * _NL, _NL)] = incl - tot[c] + carry_e
            carry_e = carry_e + jnp.sum(tot[c])
        pltpu.sync_copy(st1_v.at[pl.ds(0, _B)], ne_hbm)
        pltpu.sync_copy(st2_v.at[pl.ds(0, _B)], cume_hbm)
        pltpu.sync_copy(rn_hbm, re_all)

        def nacc(wp, carry):
            return tuple(carry[c] + re_all[wp, pl.ds(c * _NL, _NL)]
                         for c in range(4))
        ntot = lax.fori_loop(0, _NW, nacc, tuple([zeros] * 4))
        for c in range(4):
            st1_v[pl.ds(c * _NL, _NL)] = ntot[c]
        ii = _IOTA()
        for c in range(4):
            gidx = ii + (c * _NL - 1)
            shifted = plsc.load_gather(st1_v, [jnp.maximum(gidx, 0)])
            st2_v[pl.ds(c * _NL, _NL)] = jnp.where(gidx < 0, 0, shifted)
        pltpu.sync_copy(st2_v.at[pl.ds(0, _B)], delta_hbm)

    nrows = _S // _NSTEP

    def prow(r, _):
        slot = lax.rem(r, _NRING)

        @pl.when(r >= _NRING)
        def _():
            pltpu.make_async_copy(
                sni_hbm.at[pl.ds(0, 3 * _RW)], drain_v, sem).wait()

        for j in range(_NSTEP):
            idx = lane * _S + r * _NSTEP + j
            k = plsc.load_gather(e2g_c, [idx])
            d = plsc.load_gather(cur_v, [k, lane])
            plsc.store_scatter(cur_v, [k, lane], d + 1)
            dring[slot, pl.ds(j * _NL, _NL)] = d
            niring[slot, pl.ds(j * _NL, _NL)] = plsc.load_gather(nin_c, [idx])
            noring[slot, pl.ds(j * _NL, _NL)] = plsc.load_gather(nout_c, [idx])
            rring[slot, pl.ds(j * _NL, _NL)] = plsc.load_gather(rel_c, [idx])
        pltpu.async_copy(niring.at[slot], sni_hbm.at[dring.at[slot]], sem)
        pltpu.async_copy(noring.at[slot], sno_hbm.at[dring.at[slot]], sem)
        pltpu.async_copy(rring.at[slot], sr_hbm.at[dring.at[slot]], sem)
        return 0
    lax.fori_loop(0, nrows, prow, 0)

    def dr(i, _):
        pltpu.make_async_copy(
            sni_hbm.at[pl.ds(0, 3 * _RW)], drain_v, sem).wait()
        return 0
    lax.fori_loop(0, _NRING, dr, 0)


_RPAD = 12512
_PER = 14 * _RW
_NPER = _CH // _PER
_NRG = 4


@functools.partial(
    pl.kernel,
    out_type=[
        jax.ShapeDtypeStruct((_EPAD,), jnp.int32),
        jax.ShapeDtypeStruct((_EPAD,), jnp.int32),
        jax.ShapeDtypeStruct((_EPAD,), jnp.int32),
        jax.ShapeDtypeStruct((_EPAD,), jnp.float32),
    ],
    mesh=_MESH,
    compiler_params=pltpu.CompilerParams(needs_layout_passes=False),
    scratch_types=[
        pltpu.VMEM((_NPAD,), jnp.int32),
        pltpu.VMEM((_RPAD,), jnp.int32),
        pltpu.VMEM((_PER,), jnp.int32),
        pltpu.VMEM((_PER,), jnp.int32),
        pltpu.VMEM((_NRG, _RW), jnp.int32),
        pltpu.VMEM((_NRG, _RW), jnp.int32),
        pltpu.VMEM((_PER,), jnp.int32),
        pltpu.VMEM((_PER,), jnp.int32),
        pltpu.VMEM((_PER,), jnp.int32),
        pltpu.VMEM((_PER,), jnp.float32),
        pltpu.VMEM((_NRG * 3, _RW), jnp.float32),
        pltpu.VMEM((_NRG * 3, _RW), jnp.float32),
        pltpu.VMEM((_RW,), jnp.float32),
        pltpu.SemaphoreType.DMA,
    ],
)
def _pass_cd(sni_hbm, sno_hbm, b2r_hbm, rt_hbm, px_hbm, py_hbm, pz_hbm,
             ti_hbm, to_hbm, sd_hbm, dsq_hbm,
             b2r_v, rt_v, sni_sl, sno_sl, ixin_v, ixout_v,
             ti_b, to_b, sd_b, dsq_b, rin_v, rout_v, drain_v, sem):
    w = _wid()
    pltpu.sync_copy(b2r_hbm, b2r_v)
    pltpu.sync_copy(rt_hbm, rt_v)

    def distchunk(c):
        slot = lax.rem(c, _NRG)
        for _i in range(6):
            pltpu.make_async_copy(
                px_hbm.at[pl.ds(0, _RW)], drain_v, sem).wait()
        for v in range(_RW // _NL):
            acc = jnp.zeros((_NL,), jnp.float32)
            for comp in range(3):
                xin = rin_v[slot * 3 + comp, pl.ds(v * _NL, _NL)]
                xout = rout_v[slot * 3 + comp, pl.ds(v * _NL, _NL)]
                d = xin - xout
                acc = acc + d * d
            dsq_b[pl.ds(c * _RW + v * _NL, _NL)] = acc

    def period(p, _):
        off = w * _CH + p * _PER
        pltpu.sync_copy(sni_hbm.at[pl.ds(off, _PER)], sni_sl)
        pltpu.sync_copy(sno_hbm.at[pl.ds(off, _PER)], sno_sl)

        def chunk(c, _):
            slot = lax.rem(c, _NRG)

            @pl.when(c >= 2)
            def _():
                distchunk(c - 2)
            for v in range(_RW // _NL):
                s = c * _RW + v * _NL
                ni = sni_sl[pl.ds(s, _NL)]
                no = sno_sl[pl.ds(s, _NL)]
                ixin_v[slot, pl.ds(v * _NL, _NL)] = ni
                ixout_v[slot, pl.ds(v * _NL, _NL)] = no
                ri = plsc.load_gather(b2r_v, [ni])
                ro = plsc.load_gather(b2r_v, [no])
                ti = plsc.load_gather(rt_v, [ri])
                to = plsc.load_gather(rt_v, [ro])
                sd = jnp.minimum(jnp.abs(ri - ro), _MAXD)
                ti_b[pl.ds(s, _NL)] = ti
                to_b[pl.ds(s, _NL)] = to
                sd_b[pl.ds(s, _NL)] = sd
            idx_in = ixin_v.at[slot]
            idx_out = ixout_v.at[slot]
            pltpu.async_copy(px_hbm.at[idx_in], rin_v.at[slot * 3], sem)
            pltpu.async_copy(py_hbm.at[idx_in], rin_v.at[slot * 3 + 1], sem)
            pltpu.async_copy(pz_hbm.at[idx_in], rin_v.at[slot * 3 + 2], sem)
            pltpu.async_copy(px_hbm.at[idx_out], rout_v.at[slot * 3], sem)
            pltpu.async_copy(py_hbm.at[idx_out], rout_v.at[slot * 3 + 1], sem)
            pltpu.async_copy(pz_hbm.at[idx_out], rout_v.at[slot * 3 + 2], sem)
            return 0
        lax.fori_loop(0, 14, chunk, 0)
        distchunk(12)
        distchunk(13)
        pltpu.sync_copy(ti_b, ti_hbm.at[pl.ds(off, _PER)])
        pltpu.sync_copy(to_b, to_hbm.at[pl.ds(off, _PER)])
        pltpu.sync_copy(sd_b, sd_hbm.at[pl.ds(off, _PER)])
        pltpu.sync_copy(dsq_b, dsq_hbm.at[pl.ds(off, _PER)])
        return 0
    lax.fori_loop(0, _NPER, period, 0)


def _feat_body(ti_ref, to_ref, r_ref, sd_ref, dist_ref, delta_ref, cume_ref,
               feat_ref, off_ref):
    col = jax.lax.broadcasted_iota(jnp.int32, (_RB, _F), 1)
    ti = ti_ref[0, 0, :][:, None]
    to = to_ref[0, 0, :][:, None]
    rr = r_ref[0, 0, :][:, None]
    sd = sd_ref[0, 0, :][:, None]
    hit = (col == ti) | (col == to + _NUM_RES) | (col == rr + 2 * _NUM_RES) | (
        col == sd + 2 * _NUM_RES + _NUM_REL)
    f = jnp.where(hit, jnp.float32(1.0), jnp.float32(0.0))
    dist = jnp.sqrt(dist_ref[0, 0, :] + jnp.float32(1e-12))
    f = jnp.where(col == _F - 1, dist[:, None], f)
    feat_ref[...] = f
    d0 = pl.program_id(0) * _RB
    dmat = d0 + jax.lax.broadcasted_iota(jnp.int32, (_RB, _B), 0)
    ge = dmat >= cume_ref[:][None, :]
    off_ref[0, 0, :] = jnp.sum(jnp.where(ge, delta_ref[:][None, :], 0), axis=1)


@jax.jit
def _feature_call(ti, to, r, sd, dist, delta, cume):
    e = ti.shape[0]
    grid = e // _RB
    row_spec = pl.BlockSpec((1, 1, _RB), lambda i: (i, 0, 0))
    small_spec = pl.BlockSpec((_B,), lambda i: (0,))
    r3 = lambda x: x.reshape(grid, 1, _RB)
    feat, off3 = pl.pallas_call(
        _feat_body,
        grid=(grid,),
        in_specs=[row_spec, row_spec, row_spec, row_spec, row_spec,
                  small_spec, small_spec],
        out_specs=[pl.BlockSpec((_RB, _F), lambda i: (i, 0)), row_spec],
        out_shape=[
            jax.ShapeDtypeStruct((e, _F), jnp.float32),
            jax.ShapeDtypeStruct((grid, 1, _RB), jnp.int32),
        ],
    )(r3(ti), r3(to), r3(r), r3(sd), r3(dist), delta, cume)
    return feat, off3.reshape(e)


def kernel(node_position, node_in, node_out, relation, node2graph,
           bead2residue, residue_type):
    e = node_in.shape[0]
    epad = _EPAD - e
    sentinel = jnp.int32(_NPAD - 1)
    nin_p = jnp.concatenate([node_in, jnp.full((epad,), sentinel, jnp.int32)])
    nout_p = jnp.concatenate([node_out, jnp.full((epad,), sentinel, jnp.int32)])
    rel_p = jnp.concatenate([relation, jnp.zeros((epad,), jnp.int32)])
    n2g_p = jnp.concatenate(
        [node2graph, jnp.full((_NPAD - _N,), jnp.int32(_B), jnp.int32)])
    e2g_p, histe, re_sums, rn_sums = _pass_a(nin_p, n2g_p)
    sni, sno, sr, num_edges, cume, delta = _pass_b(
        e2g_p, histe, re_sums, rn_sums, nin_p, nout_p, rel_p)
    b2r_p = jnp.concatenate(
        [bead2residue, jnp.zeros((_NPAD - _N,), jnp.int32)])
    rt_p = jnp.concatenate(
        [residue_type, jnp.zeros((_RPAD - residue_type.shape[0],), jnp.int32)])
    pos_p = jnp.pad(node_position, ((0, _NPAD - _N), (0, 0)))
    px_p = pos_p[:, 0]
    py_p = pos_p[:, 1]
    pz_p = pos_p[:, 2]
    ti_s, to_s, sd_s, dsq_s = _pass_cd(
        sni, sno, b2r_p, rt_p, px_p, py_p, pz_p)
    ni = sni[:e]
    no = sno[:e]
    r = sr[:e]
    feat, offsets = _feature_call(
        ti_s[:e], to_s[:e], r, sd_s[:e], dsq_s[:e], delta, cume)
    edge_list = jnp.stack([ni, no, r], axis=-1)
    return edge_list, num_edges, offsets, feat


# speedup vs baseline: 11.9871x; 1.4772x over previous
import functools

import jax
import jax.numpy as jnp
from jax import lax
from jax.experimental import pallas as pl
from jax.experimental.pallas import tpu as pltpu
from jax.experimental.pallas import tpu_sc as plsc

_NUM_RES = 21
_NUM_REL = 7
_MAXD = 10
_B = 64
_F = _NUM_RES * 2 + _NUM_REL + _MAXD + 1 + 1
_RB = 6400

_NW = 32
_NL = 16
_E = 800000
_S = 1568
_CH = _S * _NL
_EPAD = _S * _NL * _NW
_N = 50000
_NPAD = 50176
_NCH = _NPAD // _NW
_KB = 80


_MESH = plsc.VectorSubcoreMesh(core_axis_name="c", subcore_axis_name="s")
_IOTA = functools.partial(lax.broadcasted_iota, jnp.int32, (_NL,), 0)


def _wid():
    return lax.axis_index("c") * _NL + lax.axis_index("s")


@functools.partial(
    pl.kernel,
    out_type=[
        jax.ShapeDtypeStruct((_EPAD,), jnp.int32),
        jax.ShapeDtypeStruct((_NW, _KB, _NL), jnp.int32),
        jax.ShapeDtypeStruct((_NW, _KB), jnp.int32),
        jax.ShapeDtypeStruct((_NW, _KB), jnp.int32),
    ],
    mesh=_MESH,
    compiler_params=pltpu.CompilerParams(needs_layout_passes=False),
    scratch_types=[
        pltpu.VMEM((_NPAD,), jnp.int32),
        pltpu.VMEM((_CH,), jnp.int32),
        pltpu.VMEM((_CH,), jnp.int32),
        pltpu.VMEM((_KB, _NL), jnp.int32),
        pltpu.VMEM((_KB, _NL), jnp.int32),
        pltpu.VMEM((_NL, _KB), jnp.int32),
        pltpu.VMEM((_NL, _KB), jnp.int32),
        pltpu.VMEM((_KB,), jnp.int32),
        pltpu.VMEM((_KB,), jnp.int32),
    ],
)
def _pass_a(nin_hbm, n2g_hbm, e2g_hbm, histe_hbm, re_hbm, rn_hbm,
            n2g_v, nin_v, e2g_v, he_v, hn_v, heT_v, hnT_v, re_v, rn_v):
    w = _wid()
    lane = _IOTA()
    ones = jnp.ones((_NL,), jnp.int32)
    zeros = jnp.zeros((_NL,), jnp.int32)
    pltpu.sync_copy(n2g_hbm, n2g_v)
    pltpu.sync_copy(nin_hbm.at[pl.ds(w * _CH, _CH)], nin_v)

    def zrow(k, _):
        he_v[k, :] = zeros
        hn_v[k, :] = zeros
        return 0
    lax.fori_loop(0, _KB, zrow, 0)

    def estep(s, _):
        idx = lane * _S + s
        ni = plsc.load_gather(nin_v, [idx])
        k = plsc.load_gather(n2g_v, [ni])
        plsc.store_scatter(e2g_v, [idx], k)
        plsc.addupdate_scatter(he_v, [k, lane], ones)
        return 0
    lax.fori_loop(0, _S, estep, 0)
    pltpu.sync_copy(e2g_v, e2g_hbm.at[pl.ds(w * _CH, _CH)])

    def nstep(s, _):
        k = n2g_v[pl.ds(w * _NCH + s * _NL, _NL)]
        plsc.addupdate_scatter(hn_v, [k, lane], ones)
        return 0
    lax.fori_loop(0, _NCH // _NL, nstep, 0)

    def tpose(k, _):
        kv = jnp.full((_NL,), k, jnp.int32)
        plsc.store_scatter(heT_v, [lane, kv], he_v[k, :])
        plsc.store_scatter(hnT_v, [lane, kv], hn_v[k, :])
        return 0
    lax.fori_loop(0, _KB, tpose, 0)
    for c in range(_KB // _NL):
        acc_e = jnp.zeros((_NL,), jnp.int32)
        acc_n = jnp.zeros((_NL,), jnp.int32)
        for l in range(_NL):
            acc_e = acc_e + heT_v[l, pl.ds(c * _NL, _NL)]
            acc_n = acc_n + hnT_v[l, pl.ds(c * _NL, _NL)]
        re_v[pl.ds(c * _NL, _NL)] = acc_e
        rn_v[pl.ds(c * _NL, _NL)] = acc_n
    pltpu.sync_copy(he_v, histe_hbm.at[w])
    pltpu.sync_copy(re_v, re_hbm.at[w])
    pltpu.sync_copy(rn_v, rn_hbm.at[w])


_NRING = 16
_RW = 128
_NSTEP = _RW // _NL
_PER = 14 * _RW
_NPER = _CH // _PER


@functools.partial(
    pl.kernel,
    out_type=[
        jax.ShapeDtypeStruct((_EPAD,), jnp.int32),
        jax.ShapeDtypeStruct((_B,), jnp.int32),
        jax.ShapeDtypeStruct((_B,), jnp.int32),
        jax.ShapeDtypeStruct((_B,), jnp.int32),
    ],
    mesh=_MESH,
    compiler_params=pltpu.CompilerParams(needs_layout_passes=False),
    scratch_types=[
        pltpu.VMEM((_NW, _KB), jnp.int32),
        pltpu.VMEM((_KB, _NL), jnp.int32),
        pltpu.VMEM((_KB, _NL), jnp.int32),
        pltpu.VMEM((_KB,), jnp.int32),
        pltpu.VMEM((_KB,), jnp.int32),
        pltpu.VMEM((_KB,), jnp.int32),
        pltpu.VMEM((_CH,), jnp.int32),
        pltpu.VMEM((_NRING, _RW), jnp.int32),
        pltpu.VMEM((_NRING, _RW), jnp.int32),
        pltpu.VMEM((_RW,), jnp.int32),
        pltpu.SemaphoreType.DMA,
    ],
)
def _pass_b(e2g_hbm, histe_hbm, re_hbm, rn_hbm,
            ord_hbm, ne_hbm, cume_hbm, delta_hbm,
            re_all, he_v, cur_v, pb_v, st1_v, st2_v,
            e2g_c, dring, ering, drain_v, sem):
    w = _wid()
    lane = _IOTA()
    pltpu.sync_copy(re_hbm, re_all)
    pltpu.sync_copy(histe_hbm.at[w], he_v)
    pltpu.sync_copy(e2g_hbm.at[pl.ds(w * _CH, _CH)], e2g_c)

    nch = _KB // _NL
    zeros = jnp.zeros((_NL,), jnp.int32)

    def wacc(wp, carry):
        tot = list(carry[:nch])
        spv = list(carry[nch:])
        for c in range(nch):
            row = re_all[wp, pl.ds(c * _NL, _NL)]
            tot[c] = tot[c] + row
            spv[c] = spv[c] + jnp.where(wp < w, row, 0)
        return tuple(tot) + tuple(spv)
    init = tuple([zeros] * (2 * nch))
    acc = lax.fori_loop(0, _NW, wacc, init)
    tot = acc[:nch]
    spv = acc[nch:]

    carry = jnp.int32(0)
    for c in range(nch):
        incl = jnp.cumsum(tot[c])
        pb_v[pl.ds(c * _NL, _NL)] = incl - tot[c] + carry + spv[c]
        carry = carry + jnp.sum(tot[c])

    def baserow(k, _):
        row = he_v[k, :]
        incl = jnp.cumsum(row)
        pb = plsc.load_gather(pb_v, [jnp.full((_NL,), k, jnp.int32)])
        cur_v[k, :] = incl - row + pb
        return 0
    lax.fori_loop(0, _KB, baserow, 0)

    @pl.when(w == 0)
    def _():
        carry_e = jnp.int32(0)
        for c in range(4):
            st1_v[pl.ds(c * _NL, _NL)] = tot[c]
            incl = jnp.cumsum(tot[c])
            st2_v[pl.ds(c * _NL, _NL)] = incl - tot[c] + carry_e
            carry_e = carry_e + jnp.sum(tot[c])
        pltpu.sync_copy(st1_v.at[pl.ds(0, _B)], ne_hbm)
        pltpu.sync_copy(st2_v.at[pl.ds(0, _B)], cume_hbm)
        pltpu.sync_copy(rn_hbm, re_all)

        def nacc(wp, carry):
            return tuple(carry[c] + re_all[wp, pl.ds(c * _NL, _NL)]
                         for c in range(4))
        ntot = lax.fori_loop(0, _NW, nacc, tuple([zeros] * 4))
        for c in range(4):
            st1_v[pl.ds(c * _NL, _NL)] = ntot[c]
        ii = _IOTA()
        for c in range(4):
            gidx = ii + (c * _NL - 1)
            shifted = plsc.load_gather(st1_v, [jnp.maximum(gidx, 0)])
            st2_v[pl.ds(c * _NL, _NL)] = jnp.where(gidx < 0, 0, shifted)
        pltpu.sync_copy(st2_v.at[pl.ds(0, _B)], delta_hbm)

    nrows = _S // _NSTEP

    def prow(r, _):
        slot = lax.rem(r, _NRING)

        @pl.when(r >= _NRING)
        def _():
            pltpu.make_async_copy(
                ord_hbm.at[pl.ds(0, _RW)], drain_v, sem).wait()

        for j in range(_NSTEP):
            idx = lane * _S + r * _NSTEP + j
            k = plsc.load_gather(e2g_c, [idx])
            d = plsc.load_gather(cur_v, [k, lane])
            plsc.store_scatter(cur_v, [k, lane], d + 1)
            dring[slot, pl.ds(j * _NL, _NL)] = d
            ering[slot, pl.ds(j * _NL, _NL)] = idx + w * _CH
        pltpu.async_copy(ering.at[slot], ord_hbm.at[dring.at[slot]], sem)
        return 0
    lax.fori_loop(0, nrows, prow, 0)

    def dr(i, _):
        pltpu.make_async_copy(
            ord_hbm.at[pl.ds(0, _RW)], drain_v, sem).wait()
        return 0
    lax.fori_loop(0, _NRING, dr, 0)


_NRG1 = 8


@functools.partial(
    pl.kernel,
    out_type=[
        jax.ShapeDtypeStruct((_EPAD,), jnp.int32),
        jax.ShapeDtypeStruct((_EPAD,), jnp.int32),
        jax.ShapeDtypeStruct((_EPAD,), jnp.int32),
    ],
    mesh=_MESH,
    compiler_params=pltpu.CompilerParams(needs_layout_passes=False),
    scratch_types=[
        pltpu.VMEM((_PER,), jnp.int32),
        pltpu.VMEM((_NRG1, _RW), jnp.int32),
        pltpu.VMEM((_NRG1, _RW), jnp.int32),
        pltpu.VMEM((_NRG1, _RW), jnp.int32),
        pltpu.VMEM((_NRG1, _RW), jnp.int32),
        pltpu.VMEM((_PER,), jnp.int32),
        pltpu.VMEM((_PER,), jnp.int32),
        pltpu.VMEM((_PER,), jnp.int32),
        pltpu.VMEM((_RW,), jnp.int32),
        pltpu.SemaphoreType.DMA,
    ],
)
def _pass_cd1(ord_hbm, nin_hbm, nout_hbm, rel_hbm,
              sni_hbm, sno_hbm, sr_hbm,
              ord_sl, ix_v, rni_v, rno_v, rr_v,
              sni_b, sno_b, sr_b, drain_v, sem):
    w = _wid()

    def pull(c):
        slot = lax.rem(c, _NRG1)
        for _i in range(3):
            pltpu.make_async_copy(
                ord_hbm.at[pl.ds(0, _RW)], drain_v, sem).wait()
        for v in range(_RW // _NL):
            s = c * _RW + v * _NL
            sni_b[pl.ds(s, _NL)] = rni_v[slot, pl.ds(v * _NL, _NL)]
            sno_b[pl.ds(s, _NL)] = rno_v[slot, pl.ds(v * _NL, _NL)]
            sr_b[pl.ds(s, _NL)] = rr_v[slot, pl.ds(v * _NL, _NL)]

    def period(p, _):
        off = w * _CH + p * _PER
        pltpu.sync_copy(ord_hbm.at[pl.ds(off, _PER)], ord_sl)

        def chunk(c, _):
            slot = lax.rem(c, _NRG1)

            @pl.when(c >= 2)
            def _():
                pull(c - 2)
            for v in range(_RW // _NL):
                ix_v[slot, pl.ds(v * _NL, _NL)] = ord_sl[
                    pl.ds(c * _RW + v * _NL, _NL)]
            idx = ix_v.at[slot]
            pltpu.async_copy(nin_hbm.at[idx], rni_v.at[slot], sem)
            pltpu.async_copy(nout_hbm.at[idx], rno_v.at[slot], sem)
            pltpu.async_copy(rel_hbm.at[idx], rr_v.at[slot], sem)
            return 0
        lax.fori_loop(0, 14, chunk, 0)
        pull(12)
        pull(13)
        pltpu.sync_copy(sni_b, sni_hbm.at[pl.ds(off, _PER)])
        pltpu.sync_copy(sno_b, sno_hbm.at[pl.ds(off, _PER)])
        pltpu.sync_copy(sr_b, sr_hbm.at[pl.ds(off, _PER)])
        return 0
    lax.fori_loop(0, _NPER, period, 0)


_RPAD = 12512
_NRG = 4


@functools.partial(
    pl.kernel,
    out_type=[
        jax.ShapeDtypeStruct((_EPAD,), jnp.int32),
        jax.ShapeDtypeStruct((_EPAD,), jnp.int32),
        jax.ShapeDtypeStruct((_EPAD,), jnp.int32),
        jax.ShapeDtypeStruct((_EPAD,), jnp.float32),
    ],
    mesh=_MESH,
    compiler_params=pltpu.CompilerParams(needs_layout_passes=False),
    scratch_types=[
        pltpu.VMEM((_NPAD,), jnp.int32),
        pltpu.VMEM((_RPAD,), jnp.int32),
        pltpu.VMEM((_PER,), jnp.int32),
        pltpu.VMEM((_PER,), jnp.int32),
        pltpu.VMEM((_NRG, _RW), jnp.int32),
        pltpu.VMEM((_NRG, _RW), jnp.int32),
        pltpu.VMEM((_PER,), jnp.int32),
        pltpu.VMEM((_PER,), jnp.int32),
        pltpu.VMEM((_PER,), jnp.int32),
        pltpu.VMEM((_PER,), jnp.float32),
        pltpu.VMEM((_NRG * 3, _RW), jnp.float32),
        pltpu.VMEM((_NRG * 3, _RW), jnp.float32),
        pltpu.VMEM((_RW,), jnp.float32),
        pltpu.SemaphoreType.DMA,
    ],
)
def _pass_cd(sni_hbm, sno_hbm, b2r_hbm, rt_hbm, px_hbm, py_hbm, pz_hbm,
             ti_hbm, to_hbm, sd_hbm, dsq_hbm,
             b2r_v, rt_v, sni_sl, sno_sl, ixin_v, ixout_v,
             ti_b, to_b, sd_b, dsq_b, rin_v, rout_v, drain_v, sem):
    w = _wid()
    pltpu.sync_copy(b2r_hbm, b2r_v)
    pltpu.sync_copy(rt_hbm, rt_v)

    def distchunk(c):
        slot = lax.rem(c, _NRG)
        for _i in range(6):
            pltpu.make_async_copy(
                px_hbm.at[pl.ds(0, _RW)], drain_v, sem).wait()
        for v in range(_RW // _NL):
            acc = jnp.zeros((_NL,), jnp.float32)
            for comp in range(3):
                xin = rin_v[slot * 3 + comp, pl.ds(v * _NL, _NL)]
                xout = rout_v[slot * 3 + comp, pl.ds(v * _NL, _NL)]
                d = xin - xout
                acc = acc + d * d
            dsq_b[pl.ds(c * _RW + v * _NL, _NL)] = acc

    def period(p, _):
        off = w * _CH + p * _PER
        pltpu.sync_copy(sni_hbm.at[pl.ds(off, _PER)], sni_sl)
        pltpu.sync_copy(sno_hbm.at[pl.ds(off, _PER)], sno_sl)

        def chunk(c, _):
            slot = lax.rem(c, _NRG)

            @pl.when(c >= 2)
            def _():
                distchunk(c - 2)
            for v in range(_RW // _NL):
                s = c * _RW + v * _NL
                ni = sni_sl[pl.ds(s, _NL)]
                no = sno_sl[pl.ds(s, _NL)]
                ixin_v[slot, pl.ds(v * _NL, _NL)] = ni
                ixout_v[slot, pl.ds(v * _NL, _NL)] = no
                ri = plsc.load_gather(b2r_v, [ni])
                ro = plsc.load_gather(b2r_v, [no])
                ti = plsc.load_gather(rt_v, [ri])
                to = plsc.load_gather(rt_v, [ro])
                sd = jnp.minimum(jnp.abs(ri - ro), _MAXD)
                ti_b[pl.ds(s, _NL)] = ti
                to_b[pl.ds(s, _NL)] = to
                sd_b[pl.ds(s, _NL)] = sd
            idx_in = ixin_v.at[slot]
            idx_out = ixout_v.at[slot]
            pltpu.async_copy(px_hbm.at[idx_in], rin_v.at[slot * 3], sem)
            pltpu.async_copy(py_hbm.at[idx_in], rin_v.at[slot * 3 + 1], sem)
            pltpu.async_copy(pz_hbm.at[idx_in], rin_v.at[slot * 3 + 2], sem)
            pltpu.async_copy(px_hbm.at[idx_out], rout_v.at[slot * 3], sem)
            pltpu.async_copy(py_hbm.at[idx_out], rout_v.at[slot * 3 + 1], sem)
            pltpu.async_copy(pz_hbm.at[idx_out], rout_v.at[slot * 3 + 2], sem)
            return 0
        lax.fori_loop(0, 14, chunk, 0)
        distchunk(12)
        distchunk(13)
        pltpu.sync_copy(ti_b, ti_hbm.at[pl.ds(off, _PER)])
        pltpu.sync_copy(to_b, to_hbm.at[pl.ds(off, _PER)])
        pltpu.sync_copy(sd_b, sd_hbm.at[pl.ds(off, _PER)])
        pltpu.sync_copy(dsq_b, dsq_hbm.at[pl.ds(off, _PER)])
        return 0
    lax.fori_loop(0, _NPER, period, 0)


def _feat_body(ti_ref, to_ref, r_ref, sd_ref, dist_ref, delta_ref, cume_ref,
               feat_ref, off_ref):
    col = jax.lax.broadcasted_iota(jnp.int32, (_RB, _F), 1)
    ti = ti_ref[0, 0, :][:, None]
    to = to_ref[0, 0, :][:, None]
    rr = r_ref[0, 0, :][:, None]
    sd = sd_ref[0, 0, :][:, None]
    hit = (col == ti) | (col == to + _NUM_RES) | (col == rr + 2 * _NUM_RES) | (
        col == sd + 2 * _NUM_RES + _NUM_REL)
    f = jnp.where(hit, jnp.float32(1.0), jnp.float32(0.0))
    dist = jnp.sqrt(dist_ref[0, 0, :] + jnp.float32(1e-12))
    f = jnp.where(col == _F - 1, dist[:, None], f)
    feat_ref[...] = f
    d0 = pl.program_id(0) * _RB
    dmat = d0 + jax.lax.broadcasted_iota(jnp.int32, (_RB, _B), 0)
    ge = dmat >= cume_ref[:][None, :]
    off_ref[0, 0, :] = jnp.sum(jnp.where(ge, delta_ref[:][None, :], 0), axis=1)


@jax.jit
def _feature_call(ti, to, r, sd, dist, delta, cume):
    e = ti.shape[0]
    grid = e // _RB
    row_spec = pl.BlockSpec((1, 1, _RB), lambda i: (i, 0, 0))
    small_spec = pl.BlockSpec((_B,), lambda i: (0,))
    r3 = lambda x: x.reshape(grid, 1, _RB)
    feat, off3 = pl.pallas_call(
        _feat_body,
        grid=(grid,),
        in_specs=[row_spec, row_spec, row_spec, row_spec, row_spec,
                  small_spec, small_spec],
        out_specs=[pl.BlockSpec((_RB, _F), lambda i: (i, 0)), row_spec],
        out_shape=[
            jax.ShapeDtypeStruct((e, _F), jnp.float32),
            jax.ShapeDtypeStruct((grid, 1, _RB), jnp.int32),
        ],
    )(r3(ti), r3(to), r3(r), r3(sd), r3(dist), delta, cume)
    return feat, off3.reshape(e)


def kernel(node_position, node_in, node_out, relation, node2graph,
           bead2residue, residue_type):
    e = node_in.shape[0]
    epad = _EPAD - e
    sentinel = jnp.int32(_NPAD - 1)
    nin_p = jnp.concatenate([node_in, jnp.full((epad,), sentinel, jnp.int32)])
    nout_p = jnp.concatenate([node_out, jnp.full((epad,), sentinel, jnp.int32)])
    rel_p = jnp.concatenate([relation, jnp.zeros((epad,), jnp.int32)])
    n2g_p = jnp.concatenate(
        [node2graph, jnp.full((_NPAD - _N,), jnp.int32(_B), jnp.int32)])
    e2g_p, histe, re_sums, rn_sums = _pass_a(nin_p, n2g_p)
    order, num_edges, cume, delta = _pass_b(e2g_p, histe, re_sums, rn_sums)
    sni, sno, sr = _pass_cd1(order, nin_p, nout_p, rel_p)
    b2r_p = jnp.concatenate(
        [bead2residue, jnp.zeros((_NPAD - _N,), jnp.int32)])
    rt_p = jnp.concatenate(
        [residue_type, jnp.zeros((_RPAD - residue_type.shape[0],), jnp.int32)])
    pos_p = jnp.pad(node_position, ((0, _NPAD - _N), (0, 0)))
    px_p = pos_p[:, 0]
    py_p = pos_p[:, 1]
    pz_p = pos_p[:, 2]
    ti_s, to_s, sd_s, dsq_s = _pass_cd(
        sni, sno, b2r_p, rt_p, px_p, py_p, pz_p)
    ni = sni[:e]
    no = sno[:e]
    r = sr[:e]
    feat, offsets = _feature_call(
        ti_s[:e], to_s[:e], r, sd_s[:e], dsq_s[:e], delta, cume)
    edge_list = jnp.stack([ni, no, r], axis=-1)
    return edge_list, num_edges, offsets, feat
